# baseline (speedup 1.0000x reference)
import functools

import jax
import jax.numpy as jnp
from jax import lax
from jax.experimental import pallas as pl
from jax.experimental.pallas import tpu as pltpu
from jax.experimental.pallas import tpu_sc as plsc

_B, _S, _D = 4, 8192, 1024
_NC, _NS = 2, 16
_NW = _NC * _NS
_SPW = _S // _NW
_C = 8
_NCH = _SPW // _C
_NV = _C * _D // 16
_NR = 3

_mesh = plsc.VectorSubcoreMesh(core_axis_name="c", subcore_axis_name="s")

_scratch = (
    [pltpu.VMEM((_B, _C, _D), jnp.float32)] * _NR
    + [pltpu.VMEM((_C, _D), jnp.float32)] * 2
    + [pltpu.SemaphoreType.DMA] * (2 * _NR + 2)
)


@functools.partial(
    pl.kernel,
    out_type=jax.ShapeDtypeStruct((_B, _S, _D), jnp.float32),
    mesh=_mesh,
    scratch_types=_scratch,
)
def _pos_add(x_hbm, tab_hbm, out_hbm, *scr):
    xb = scr[:_NR]
    pb = scr[_NR:_NR + 2]
    ld = scr[_NR + 2:2 * _NR + 2]
    st = scr[2 * _NR + 2:3 * _NR + 2]
    ps = scr[3 * _NR + 2:]

    wid = lax.axis_index("s") * _NC + lax.axis_index("c")
    s_base = wid * _SPW

    def s0(c):
        return s_base + c * _C

    def start_load(c, k):
        pltpu.async_copy(x_hbm.at[:, pl.ds(s0(c), _C)], xb[k], ld[k])

    def wait_load(k):
        pltpu.make_async_copy(x_hbm.at[:, pl.ds(0, _C)], xb[k], ld[k]).wait()

    def start_store(c, k):
        pltpu.async_copy(xb[k], out_hbm.at[:, pl.ds(s0(c), _C)], st[k])

    def wait_store(k):
        pltpu.make_async_copy(xb[k], out_hbm.at[:, pl.ds(0, _C)], st[k]).wait()

    def start_pos(c, q):
        pltpu.async_copy(tab_hbm.at[pl.ds(s0(c), _C)], pb[q], ps[q])

    def wait_pos(q):
        pltpu.make_async_copy(tab_hbm.at[pl.ds(0, _C)], pb[q], ps[q]).wait()

    def do_add(k, q):
        buf, pos = xb[k], pb[q]

        @plsc.parallel_loop(0, _NV, unroll=2)
        def add_vec(i):
            r = i >> 6
            j = (i & 63) * 16
            v = pos[r, pl.ds(j, 16)]
            for b in range(_B):
                plsc.addupdate(buf.at[b, r, pl.ds(j, 16)], v)

    def gen_iter(c, k, q, first):
        wait_pos(q)
        wait_load(k)
        do_add(k, q)
        start_store(c, k)

        @pl.when(c + 2 < _NCH)
        def _prefetch(c=c, k=k, q=q):
            if not first:
                wait_store((k + 2) % _NR)
            start_load(c + 2, (k + 2) % _NR)
            start_pos(c + 2, q)

    start_load(0, 0)
    start_load(1, 1)
    start_pos(0, 0)
    start_pos(1, 1)

    gen_iter(0, 0, 0, first=True)
    gen_iter(1, 1, 1, first=False)

    def group_body(g, carry):
        c_lo = 2 + 6 * g
        for i in range(6):
            gen_iter(c_lo + i, (2 + i) % _NR, i % 2, first=False)
        return carry

    lax.fori_loop(0, (_NCH - 2) // 6, group_body, 0)

    for k in range(_NR):
        wait_store(k)


def kernel(x, pos_table):
    return _pos_add(x, pos_table)
